# baseline (speedup 1.0000x reference)
import functools

import jax
import jax.numpy as jnp
from jax import lax
from jax.experimental import pallas as pl
from jax.experimental.pallas import tpu as pltpu
from jax.experimental.pallas import tpu_sc as plsc

_F = 4
_V1 = 100001
_D = 32
_B = 4096
_L = 50
_NW = 32
_BPW = _B // _NW
_G = 32
_NC = _BPW // _G
_NROUND = _NC * _F

_mesh = plsc.VectorSubcoreMesh(core_axis_name="c", subcore_axis_name="s")


@functools.partial(
    pl.kernel,
    out_type=jax.ShapeDtypeStruct((_B, _F, _D), jnp.float32),
    mesh=_mesh,
    scratch_types=[
        pltpu.VMEM((_G, _L), jnp.int32),
        pltpu.VMEM((_G, _L), jnp.int32),
        pltpu.VMEM((_G * _L,), jnp.int32),
        pltpu.VMEM((_G * _L,), jnp.int32),
        pltpu.VMEM((_G * _L, _D), jnp.float32),
        pltpu.VMEM((_G * _L, _D), jnp.float32),
        pltpu.VMEM((_G, _F, _D), jnp.float32),
        pltpu.SemaphoreType.DMA,
        pltpu.SemaphoreType.DMA,
    ],
    compiler_params=pltpu.CompilerParams(use_tc_tiling_on_sc=False),
)
def _tsf_pool(w_hbm, x_hbm, out_hbm, idx0, idx1, idxf0, idxf1,
              rows0, rows1, out_v, sem0, sem1):
    wid = lax.axis_index("s") * 2 + lax.axis_index("c")
    idx = (idx0, idx1)
    idxf = (idxf0, idxf1)
    rows = (rows0, rows1)
    sems = (sem0, sem1)

    def stage(i):
        s = i % 2
        c, f = divmod(i, _F)
        b0 = wid * _BPW + c * _G
        pltpu.sync_copy(x_hbm.at[f, pl.ds(b0, _G)], idx[s])

        def _flat(g, _):
            for k in (0, 16, 32, 34):
                idxf[s][pl.ds(g * _L + k, 16)] = idx[s][g, pl.ds(k, 16)]
            return 0
        lax.fori_loop(0, _G, _flat, 0)
        return pltpu.async_copy(w_hbm.at[f].at[idxf[s]], rows[s], sems[s])

    pending = stage(0)
    for i in range(_NROUND):
        s = i % 2
        c, f = divmod(i, _F)
        desc = pending
        if i + 1 < _NROUND:
            pending = stage(i + 1)
        desc.wait()

        idx_s = idx[s]
        rows_s = rows[s]

        def _pool_one(g, _):
            r0 = g * _L
            acc0 = rows_s[r0, pl.ds(0, 16)]
            acc1 = rows_s[r0, pl.ds(16, 16)]
            for j in range(1, _L):
                acc0 = acc0 + rows_s[r0 + j, pl.ds(0, 16)]
                acc1 = acc1 + rows_s[r0 + j, pl.ds(16, 16)]
            ones = jnp.zeros((16,), jnp.float32)
            for k in range(3):
                v = idx_s[g, pl.ds(k * 16, 16)]
                ones = ones + jnp.where(v != 0, 1.0, 0.0).astype(jnp.float32)
            lane = lax.iota(jnp.int32, 16)
            v3 = idx_s[g, pl.ds(34, 16)]
            ones = ones + jnp.where((lane >= 14) & (v3 != 0), 1.0, 0.0)
            for st in (8, 4, 2, 1):
                perm = jnp.bitwise_xor(lane, st)
                ones = ones + ones.at[perm].get(mode="promise_in_bounds")
            rinv = 1.0 / ones
            out_v[g, f, pl.ds(0, 16)] = acc0 * rinv
            out_v[g, f, pl.ds(16, 16)] = acc1 * rinv
            return 0
        lax.fori_loop(0, _G, _pool_one, 0)

        if f == _F - 1:
            b0 = wid * _BPW + c * _G
            pltpu.sync_copy(out_v, out_hbm.at[pl.ds(b0, _G)])


def kernel(x, W):
    return _tsf_pool(W, x)


# speedup vs baseline: 2.2144x; 2.2144x over previous
import functools

import jax
import jax.numpy as jnp
from jax import lax
from jax.experimental import pallas as pl
from jax.experimental.pallas import tpu as pltpu
from jax.experimental.pallas import tpu_sc as plsc

_F = 4
_V1 = 100001
_D = 32
_B = 4096
_L = 50
_NW = 32
_BPW = _B // _NW
_G = 32
_NC = _BPW // _G
_NROUND = _NC * _F

_mesh = plsc.VectorSubcoreMesh(core_axis_name="c", subcore_axis_name="s")


@functools.partial(
    pl.kernel,
    out_type=jax.ShapeDtypeStruct((_B, _F, _D), jnp.float32),
    mesh=_mesh,
    scratch_types=[
        pltpu.VMEM((_G * _L,), jnp.int32),
        pltpu.VMEM((_G * _L,), jnp.int32),
        pltpu.VMEM((_G * _L, _D), jnp.float32),
        pltpu.VMEM((_G * _L, _D), jnp.float32),
        pltpu.VMEM((_G, _F, _D), jnp.float32),
        pltpu.SemaphoreType.DMA,
        pltpu.SemaphoreType.DMA,
    ],
    compiler_params=pltpu.CompilerParams(use_tc_tiling_on_sc=False),
)
def _tsf_pool(w_hbm, x_hbm, out_hbm, idx0, idx1, rows0, rows1, out_v,
              sem0, sem1):
    wid = lax.axis_index("s") * 2 + lax.axis_index("c")
    idx = (idx0, idx1)
    rows = (rows0, rows1)
    sems = (sem0, sem1)

    def stage(i):
        s = i % 2
        c, f = divmod(i, _F)
        b0 = wid * _BPW + c * _G
        src = (f * _B + b0) * _L
        pltpu.sync_copy(x_hbm.at[pl.ds(src, _G * _L)], idx[s])
        off = f * _V1
        if off:
            def _add_off(k, _):
                v = idx[s][pl.ds(k * 16, 16)]
                idx[s][pl.ds(k * 16, 16)] = v + off
                return 0
            lax.fori_loop(0, _G * _L // 16, _add_off, 0)
        return pltpu.async_copy(w_hbm.at[idx[s]], rows[s], sems[s])

    pending = stage(0)
    for i in range(_NROUND):
        s = i % 2
        c, f = divmod(i, _F)
        desc = pending
        if i + 1 < _NROUND:
            pending = stage(i + 1)
        desc.wait()

        off = f * _V1
        idx_s = idx[s]
        rows_s = rows[s]

        def _pool_one(g, _):
            r0 = g * _L
            acc0 = rows_s[r0, pl.ds(0, 16)]
            acc1 = rows_s[r0, pl.ds(16, 16)]
            for j in range(1, _L):
                acc0 = acc0 + rows_s[r0 + j, pl.ds(0, 16)]
                acc1 = acc1 + rows_s[r0 + j, pl.ds(16, 16)]
            ones = jnp.zeros((16,), jnp.float32)
            for k in range(3):
                v = idx_s[pl.ds(r0 + k * 16, 16)]
                ones = ones + jnp.where(v != off, 1.0, 0.0).astype(jnp.float32)
            lane = lax.iota(jnp.int32, 16)
            v3 = idx_s[pl.ds(r0 + 34, 16)]
            ones = ones + jnp.where((lane >= 14) & (v3 != off), 1.0, 0.0)
            for st in (8, 4, 2, 1):
                perm = jnp.bitwise_xor(lane, st)
                ones = ones + ones.at[perm].get(mode="promise_in_bounds")
            rinv = 1.0 / ones
            out_v[g, f, pl.ds(0, 16)] = acc0 * rinv
            out_v[g, f, pl.ds(16, 16)] = acc1 * rinv
            return 0
        lax.fori_loop(0, _G, _pool_one, 0)

        if f == _F - 1:
            b0 = wid * _BPW + c * _G
            pltpu.sync_copy(out_v, out_hbm.at[pl.ds(b0, _G)])


def kernel(x, W):
    x1 = x.reshape(-1)
    w1 = W.reshape(_F * _V1, _D)
    return _tsf_pool(w1, x1)
